# baseline (speedup 1.0000x reference)
import functools

import jax
import jax.numpy as jnp
import numpy as np
from jax import lax
from jax.experimental import pallas as pl
from jax.experimental.pallas import tpu as pltpu
from jax.experimental.pallas import tpu_sc as plsc

_D = 64
_ROW_BLOCK = 1000
_NC = 2
_NS = 16
_NW = _NC * _NS
_CHUNK = 128
_NGB = 6
_NTB = 3
_LAG = 5


def _mlp_body(tbl_ref, w1_ref, b1_ref, w2_ref, b2_ref, out_ref):
    x = tbl_ref[...]
    rows = lax.broadcasted_iota(jnp.int32, x.shape, 0)
    first_block = pl.program_id(0) == 0
    x = jnp.where(jnp.logical_and(first_block, rows == 0), 0.0, x)
    h = lax.dot_general(x, w1_ref[...], (((1,), (1,)), ((), ())),
                        preferred_element_type=jnp.float32)
    h = h + b1_ref[...]
    h = h * 0.5 * (1.0 + lax.erf(h * np.float32(1.0 / np.sqrt(2.0))))
    o = lax.dot_general(h, w2_ref[...], (((1,), (1,)), ((), ())),
                        preferred_element_type=jnp.float32)
    out_ref[...] = o + b2_ref[...]


def _transform_table(table, W1, b1, W2, b2):
    n = table.shape[0]
    return pl.pallas_call(
        _mlp_body,
        grid=(n // _ROW_BLOCK,),
        in_specs=[
            pl.BlockSpec((_ROW_BLOCK, _D), lambda i: (i, 0)),
            pl.BlockSpec((_D, _D), lambda i: (0, 0)),
            pl.BlockSpec((1, _D), lambda i: (0, 0)),
            pl.BlockSpec((_D, _D), lambda i: (0, 0)),
            pl.BlockSpec((1, _D), lambda i: (0, 0)),
        ],
        out_specs=pl.BlockSpec((_ROW_BLOCK, _D), lambda i: (i, 0)),
        out_shape=jax.ShapeDtypeStruct((n, _D), jnp.float32),
    )(table, W1, b1.reshape(1, _D), W2, b2.reshape(1, _D))


def _sc_gather_t(tbl2, idx4, L, B):
    nb = B // _NW
    cpl = nb // _CHUNK
    n_chunks = L * cpl
    n_main = 6 * ((n_chunks - _LAG) // 6)
    mesh = plsc.VectorSubcoreMesh(core_axis_name="c", subcore_axis_name="s")

    @functools.partial(
        pl.kernel, mesh=mesh,
        compiler_params=pltpu.CompilerParams(use_tc_tiling_on_sc=False,
                                             needs_layout_passes=False),
        out_type=jax.ShapeDtypeStruct((L, _D, B), jnp.float32),
        scratch_types=[
            pltpu.VMEM((n_chunks, _CHUNK), jnp.int32),
            pltpu.VMEM((_NGB, _CHUNK, _D), jnp.float32),
            pltpu.VMEM((_NTB, _D, _CHUNK), jnp.float32),
            pltpu.SemaphoreType.DMA((_NGB,)),
            pltpu.SemaphoreType.DMA((_NTB,)),
        ],
    )
    def k(tbl_hbm, idx_hbm, out_hbm, idx_v, gbuf, tbuf, gsem, ssem):
        wid = lax.axis_index("s") * _NC + lax.axis_index("c")
        b_base = wid * nb
        pltpu.sync_copy(idx_hbm.at[wid], idx_v)

        base_rows = [lax.iota(jnp.int32, 16) + 16 * g for g in range(8)]

        def fire_gather(j, slot):
            pltpu.async_copy(tbl_hbm.at[idx_v.at[j]],
                             gbuf.at[slot], gsem.at[slot])

        def wait_gather(slot):
            pltpu.make_async_copy(tbl_hbm.at[idx_v.at[0]],
                                  gbuf.at[slot], gsem.at[slot]).wait()

        def fire_scatter(q, slot):
            l, c = q // cpl, q % cpl
            pltpu.async_copy(
                tbuf.at[slot],
                out_hbm.at[l, :, pl.ds(b_base + c * _CHUNK, _CHUNK)],
                ssem.at[slot])

        def wait_scatter(slot):
            pltpu.make_async_copy(
                tbuf.at[slot],
                out_hbm.at[0, :, pl.ds(b_base, _CHUNK)],
                ssem.at[slot]).wait()

        def transpose(slot_g, slot_t):
            src = gbuf.at[slot_g]
            dst = tbuf.at[slot_t]

            def tb(d, carry):
                col = jnp.full((16,), d, jnp.int32)
                for g in range(8):
                    v = plsc.load_gather(src, [base_rows[g], col])
                    dst[d, pl.ds(16 * g, 16)] = v
                return carry

            lax.fori_loop(0, _D, tb, 0)

        def process(q, slot_g, slot_t, first_round):
            wait_gather(slot_g)
            if not first_round:
                wait_scatter(slot_t)
            transpose(slot_g, slot_t)
            fire_scatter(q, slot_t)

        for j in range(_LAG):
            fire_gather(j, j % _NGB)

        for u in range(6):
            j = _LAG + u
            fire_gather(j, j % _NGB)
            process(u, u % _NGB, u % _NTB, first_round=(u < _NTB))

        def body(t, carry):
            for u in range(6):
                j = _LAG + t * 6 + u
                fire_gather(j, (_LAG + u) % _NGB)
                process(j - _LAG, u, u % _NTB, first_round=False)
            return carry

        lax.fori_loop(1, (n_chunks - _LAG) // 6, body, 0)

        for q in range(n_main, n_chunks):
            j = q + _LAG
            if j < n_chunks:
                fire_gather(j, j % _NGB)
            process(q, q % _NGB, q % _NTB, first_round=False)
        for q in range(n_chunks - _NTB, n_chunks):
            wait_scatter(q % _NTB)

    return k(tbl2, idx4)


def kernel(t, table, W1, b1, W2, b2):
    tbl2 = _transform_table(table, W1, b1, W2, b2)
    B, L = t.shape
    nb = B // _NW
    cpl = nb // _CHUNK
    idx4 = (t.astype(jnp.int32).T
            .reshape(L, _NW, cpl, _CHUNK)
            .transpose(1, 0, 2, 3)
            .reshape(_NW, L * cpl, _CHUNK))
    P = _sc_gather_t(tbl2, idx4, L, B)
    return lax.transpose(P, (2, 0, 1))


# speedup vs baseline: 1.9658x; 1.9658x over previous
import functools

import jax
import jax.numpy as jnp
import numpy as np
from jax import lax
from jax.experimental import pallas as pl
from jax.experimental.pallas import tpu as pltpu
from jax.experimental.pallas import tpu_sc as plsc

_D = 64
_ROW_BLOCK = 1000
_NC = 2
_NS = 16
_NW = _NC * _NS
_CHUNK = 128
_NGB = 6
_NTB = 3
_LAG = 5


def _mlp_body(tbl_ref, w1_ref, b1_ref, w2_ref, b2_ref, out_ref):
    x = tbl_ref[...]
    rows = lax.broadcasted_iota(jnp.int32, x.shape, 0)
    first_block = pl.program_id(0) == 0
    x = jnp.where(jnp.logical_and(first_block, rows == 0), 0.0, x)
    h = lax.dot_general(x, w1_ref[...], (((1,), (1,)), ((), ())),
                        preferred_element_type=jnp.float32)
    h = h + b1_ref[...]
    h = h * 0.5 * (1.0 + lax.erf(h * np.float32(1.0 / np.sqrt(2.0))))
    o = lax.dot_general(h, w2_ref[...], (((1,), (1,)), ((), ())),
                        preferred_element_type=jnp.float32)
    out_ref[...] = o + b2_ref[...]


def _transform_table(table, W1, b1, W2, b2):
    n = table.shape[0]
    return pl.pallas_call(
        _mlp_body,
        grid=(n // _ROW_BLOCK,),
        in_specs=[
            pl.BlockSpec((_ROW_BLOCK, _D), lambda i: (i, 0)),
            pl.BlockSpec((_D, _D), lambda i: (0, 0)),
            pl.BlockSpec((1, _D), lambda i: (0, 0)),
            pl.BlockSpec((_D, _D), lambda i: (0, 0)),
            pl.BlockSpec((1, _D), lambda i: (0, 0)),
        ],
        out_specs=pl.BlockSpec((_ROW_BLOCK, _D), lambda i: (i, 0)),
        out_shape=jax.ShapeDtypeStruct((n, _D), jnp.float32),
    )(table, W1, b1.reshape(1, _D), W2, b2.reshape(1, _D))


def _sc_gather_t(tbl2, idx4, L, B):
    nb = B // _NW
    cpl = nb // _CHUNK
    n_chunks = L * cpl
    n_main = 6 * ((n_chunks - _LAG) // 6)
    mesh = plsc.VectorSubcoreMesh(core_axis_name="c", subcore_axis_name="s")

    @functools.partial(
        pl.kernel, mesh=mesh,
        compiler_params=pltpu.CompilerParams(use_tc_tiling_on_sc=False,
                                             needs_layout_passes=False),
        out_type=jax.ShapeDtypeStruct((L * _D, B), jnp.float32),
        scratch_types=[
            pltpu.VMEM((n_chunks, _CHUNK), jnp.int32),
            pltpu.VMEM((_NGB, _CHUNK, _D), jnp.float32),
            pltpu.VMEM((_NTB, _D, _CHUNK), jnp.float32),
            pltpu.SemaphoreType.DMA((_NGB,)),
            pltpu.SemaphoreType.DMA((_NTB,)),
        ],
    )
    def k(tbl_hbm, idx_hbm, out_hbm, idx_v, gbuf, tbuf, gsem, ssem):
        wid = lax.axis_index("s") * _NC + lax.axis_index("c")
        b_base = wid * nb
        pltpu.sync_copy(idx_hbm.at[wid], idx_v)

        iota16 = lax.iota(jnp.int32, 16)
        cols_base = [iota16 + 16 * bd for bd in range(4)]

        def fire_gather(j, slot):
            pltpu.async_copy(tbl_hbm.at[idx_v.at[j]],
                             gbuf.at[slot], gsem.at[slot])

        def wait_gather(slot):
            pltpu.make_async_copy(tbl_hbm.at[idx_v.at[0]],
                                  gbuf.at[slot], gsem.at[slot]).wait()

        def fire_scatter(q, slot):
            l, c = q // cpl, q % cpl
            pltpu.async_copy(
                tbuf.at[slot],
                out_hbm.at[pl.ds(l * _D, _D), pl.ds(b_base + c * _CHUNK, _CHUNK)],
                ssem.at[slot])

        def wait_scatter(slot):
            pltpu.make_async_copy(
                tbuf.at[slot],
                out_hbm.at[pl.ds(0, _D), pl.ds(b_base, _CHUNK)],
                ssem.at[slot]).wait()

        def transpose(slot_g, slot_t):
            src = gbuf.at[slot_g]
            dst = tbuf.at[slot_t]

            def tk(kk, carry):
                rot = lax.rem(iota16 + kk, 16)
                for bb in range(8):
                    rows = rot + 16 * bb
                    for bd in range(4):
                        v = plsc.load_gather(src, [rows, cols_base[bd]])
                        plsc.store_scatter(dst, [cols_base[bd], rows], v)
                return carry

            lax.fori_loop(0, 16, tk, 0)

        def process(q, slot_g, slot_t, first_round):
            wait_gather(slot_g)
            if not first_round:
                wait_scatter(slot_t)
            transpose(slot_g, slot_t)
            fire_scatter(q, slot_t)

        for j in range(_LAG):
            fire_gather(j, j % _NGB)

        for u in range(6):
            j = _LAG + u
            fire_gather(j, j % _NGB)
            process(u, u % _NGB, u % _NTB, first_round=(u < _NTB))

        def body(t, carry):
            for u in range(6):
                j = _LAG + t * 6 + u
                fire_gather(j, (_LAG + u) % _NGB)
                process(j - _LAG, u, u % _NTB, first_round=False)
            return carry

        lax.fori_loop(1, (n_chunks - _LAG) // 6, body, 0)

        for q in range(n_main, n_chunks):
            j = q + _LAG
            if j < n_chunks:
                fire_gather(j, j % _NGB)
            process(q, q % _NGB, q % _NTB, first_round=False)
        for q in range(n_chunks - _NTB, n_chunks):
            wait_scatter(q % _NTB)

    return k(tbl2, idx4)


def kernel(t, table, W1, b1, W2, b2):
    tbl2 = _transform_table(table, W1, b1, W2, b2)
    B, L = t.shape
    nb = B // _NW
    cpl = nb // _CHUNK
    idx4 = (t.astype(jnp.int32).T
            .reshape(L, _NW, cpl, _CHUNK)
            .transpose(1, 0, 2, 3)
            .reshape(_NW, L * cpl, _CHUNK))
    P = _sc_gather_t(tbl2, idx4, L, B).reshape(L, _D, B)
    return lax.transpose(P, (2, 0, 1))


# speedup vs baseline: 2.9281x; 1.4895x over previous
import functools

import jax
import jax.numpy as jnp
import numpy as np
from jax import lax
from jax.experimental import pallas as pl
from jax.experimental.pallas import tpu as pltpu
from jax.experimental.pallas import tpu_sc as plsc

_D = 64
_ROW_BLOCK = 1000
_NC = 2
_NS = 16
_NW = _NC * _NS
_CHUNK = 128
_NGB = 4
_NTB = 4
_LAG = 3
_BLK = 4


def _mlp_body(tbl_ref, w1_ref, b1_ref, w2_ref, b2_ref, out_ref):
    x = tbl_ref[...]
    rows = lax.broadcasted_iota(jnp.int32, x.shape, 0)
    first_block = pl.program_id(0) == 0
    x = jnp.where(jnp.logical_and(first_block, rows == 0), 0.0, x)
    h = lax.dot_general(x, w1_ref[...], (((1,), (1,)), ((), ())),
                        preferred_element_type=jnp.float32)
    h = h + b1_ref[...]
    h = h * 0.5 * (1.0 + lax.erf(h * np.float32(1.0 / np.sqrt(2.0))))
    o = lax.dot_general(h, w2_ref[...], (((1,), (1,)), ((), ())),
                        preferred_element_type=jnp.float32)
    o = o + b2_ref[...]
    out_ref[...] = jnp.concatenate([o, o], axis=1)


def _transform_table(table, W1, b1, W2, b2):
    n = table.shape[0]
    return pl.pallas_call(
        _mlp_body,
        grid=(n // _ROW_BLOCK,),
        in_specs=[
            pl.BlockSpec((_ROW_BLOCK, _D), lambda i: (i, 0)),
            pl.BlockSpec((_D, _D), lambda i: (0, 0)),
            pl.BlockSpec((1, _D), lambda i: (0, 0)),
            pl.BlockSpec((_D, _D), lambda i: (0, 0)),
            pl.BlockSpec((1, _D), lambda i: (0, 0)),
        ],
        out_specs=pl.BlockSpec((_ROW_BLOCK, 2 * _D), lambda i: (i, 0)),
        out_shape=jax.ShapeDtypeStruct((n, 2 * _D), jnp.float32),
    )(table, W1, b1.reshape(1, _D), W2, b2.reshape(1, _D))


def _sc_gather_t(tbl2, idx4, L, B):
    nb = B // _NW
    cpl = nb // _CHUNK
    n_chunks = L * cpl
    n_main = _BLK * ((n_chunks - _LAG) // _BLK)
    mesh = plsc.VectorSubcoreMesh(core_axis_name="c", subcore_axis_name="s")

    @functools.partial(
        pl.kernel, mesh=mesh,
        compiler_params=pltpu.CompilerParams(needs_layout_passes=False),
        out_type=jax.ShapeDtypeStruct((L * _D, B), jnp.float32),
        scratch_types=[
            pltpu.VMEM((n_chunks, _CHUNK), jnp.int32),
            pltpu.VMEM((_NGB, _CHUNK, 2 * _D), jnp.float32),
            pltpu.VMEM((_NTB, _D, _CHUNK), jnp.float32),
            pltpu.SemaphoreType.DMA((_NGB,)),
            pltpu.SemaphoreType.DMA((_NTB,)),
        ],
    )
    def k(tbl_hbm, idx_hbm, out_hbm, idx_v, gbuf, tbuf, gsem, ssem):
        wid = lax.axis_index("s") * _NC + lax.axis_index("c")
        b_base = wid * nb
        pltpu.sync_copy(idx_hbm.at[wid], idx_v)

        iota16 = lax.iota(jnp.int32, 16)
        cols_base = [iota16 + 16 * bd for bd in range(4)]

        def fire_gather(j, slot):
            pltpu.async_copy(tbl_hbm.at[idx_v.at[j]],
                             gbuf.at[slot], gsem.at[slot])

        def wait_gather(slot):
            pltpu.make_async_copy(tbl_hbm.at[idx_v.at[0]],
                                  gbuf.at[slot], gsem.at[slot]).wait()

        def fire_scatter(q, slot):
            l, c = q // cpl, q % cpl
            pltpu.async_copy(
                tbuf.at[slot],
                out_hbm.at[pl.ds(l * _D, _D), pl.ds(b_base + c * _CHUNK, _CHUNK)],
                ssem.at[slot])

        def wait_scatter(slot):
            pltpu.make_async_copy(
                tbuf.at[slot],
                out_hbm.at[pl.ds(0, _D), pl.ds(b_base, _CHUNK)],
                ssem.at[slot]).wait()

        def transpose(slot_g, slot_t):
            src = gbuf.at[slot_g]
            dst = tbuf.at[slot_t]

            def tk(kk, carry):
                rot = lax.rem(iota16 + kk, 16)
                for bb in range(8):
                    rows = rot + 16 * bb
                    for bd in range(4):
                        v = plsc.load_gather(src, [rows, cols_base[bd]])
                        plsc.store_scatter(dst, [cols_base[bd], rows], v)
                return carry

            lax.fori_loop(0, 16, tk, 0)

        def process(q, slot_g, slot_t, first_round):
            wait_gather(slot_g)
            if not first_round:
                wait_scatter(slot_t)
            transpose(slot_g, slot_t)
            fire_scatter(q, slot_t)

        for j in range(_LAG):
            fire_gather(j, j % _NGB)

        for u in range(_BLK):
            j = _LAG + u
            fire_gather(j, j % _NGB)
            process(u, u % _NGB, u % _NTB, first_round=(u < _NTB))

        def body(t, carry):
            for u in range(_BLK):
                j = _LAG + t * _BLK + u
                fire_gather(j, (_LAG + u) % _NGB)
                process(j - _LAG, u % _NGB, u % _NTB, first_round=False)
            return carry

        lax.fori_loop(1, (n_chunks - _LAG) // _BLK, body, 0)

        for q in range(n_main, n_chunks):
            j = q + _LAG
            if j < n_chunks:
                fire_gather(j, j % _NGB)
            process(q, q % _NGB, q % _NTB, first_round=False)
        for q in range(n_chunks - _NTB, n_chunks):
            wait_scatter(q % _NTB)

    return k(tbl2, idx4)


def kernel(t, table, W1, b1, W2, b2):
    tbl2 = _transform_table(table, W1, b1, W2, b2)
    B, L = t.shape
    nb = B // _NW
    cpl = nb // _CHUNK
    idx4 = (t.astype(jnp.int32).T
            .reshape(L, _NW, cpl, _CHUNK)
            .transpose(1, 0, 2, 3)
            .reshape(_NW, L * cpl, _CHUNK))
    P = _sc_gather_t(tbl2, idx4, L, B).reshape(L, _D, B)
    return lax.transpose(P, (2, 0, 1))


# speedup vs baseline: 3.2267x; 1.1020x over previous
import functools

import jax
import jax.numpy as jnp
import numpy as np
from jax import lax
from jax.experimental import pallas as pl
from jax.experimental.pallas import tpu as pltpu
from jax.experimental.pallas import tpu_sc as plsc

_D = 64
_ROW_BLOCK = 4000
_NC = 2
_NS = 16
_NW = _NC * _NS
_CHUNK = 128
_NGB = 4
_NTB = 4
_LAG = 3
_BLK = 4


def _mlp_body(tbl_ref, w1_ref, b1_ref, w2_ref, b2_ref, out_ref):
    x = tbl_ref[...]
    rows = lax.broadcasted_iota(jnp.int32, x.shape, 0)
    first_block = pl.program_id(0) == 0
    x = jnp.where(jnp.logical_and(first_block, rows == 0), 0.0, x)
    h = lax.dot_general(x, w1_ref[...], (((1,), (1,)), ((), ())),
                        preferred_element_type=jnp.float32)
    h = h + b1_ref[...]
    h = h * 0.5 * (1.0 + lax.erf(h * np.float32(1.0 / np.sqrt(2.0))))
    o = lax.dot_general(h, w2_ref[...], (((1,), (1,)), ((), ())),
                        preferred_element_type=jnp.float32)
    o = o + b2_ref[...]
    out_ref[...] = jnp.concatenate([o, o], axis=1)


def _transform_table(table, W1, b1, W2, b2):
    n = table.shape[0]
    return pl.pallas_call(
        _mlp_body,
        grid=(n // _ROW_BLOCK,),
        in_specs=[
            pl.BlockSpec((_ROW_BLOCK, _D), lambda i: (i, 0)),
            pl.BlockSpec((_D, _D), lambda i: (0, 0)),
            pl.BlockSpec((1, _D), lambda i: (0, 0)),
            pl.BlockSpec((_D, _D), lambda i: (0, 0)),
            pl.BlockSpec((1, _D), lambda i: (0, 0)),
        ],
        out_specs=pl.BlockSpec((_ROW_BLOCK, 2 * _D), lambda i: (i, 0)),
        out_shape=jax.ShapeDtypeStruct((n, 2 * _D), jnp.float32),
    )(table, W1, b1.reshape(1, _D), W2, b2.reshape(1, _D))


def _sc_gather_t(tbl2, idx4, L, B):
    nb = B // _NW
    cpl = nb // _CHUNK
    n_chunks = L * cpl
    n_main = _BLK * ((n_chunks - _LAG) // _BLK)
    mesh = plsc.VectorSubcoreMesh(core_axis_name="c", subcore_axis_name="s")

    @functools.partial(
        pl.kernel, mesh=mesh,
        compiler_params=pltpu.CompilerParams(needs_layout_passes=False),
        out_type=jax.ShapeDtypeStruct((L * _D, B), jnp.float32),
        scratch_types=[
            pltpu.VMEM((n_chunks, _CHUNK), jnp.int32),
            pltpu.VMEM((_NGB, _CHUNK, 2 * _D), jnp.float32),
            pltpu.VMEM((_NTB, _D, _CHUNK), jnp.float32),
            pltpu.SemaphoreType.DMA((_NGB,)),
            pltpu.SemaphoreType.DMA((_NTB,)),
        ],
    )
    def k(tbl_hbm, idx_hbm, out_hbm, idx_v, gbuf, tbuf, gsem, ssem):
        wid = lax.axis_index("s") * _NC + lax.axis_index("c")
        b_base = wid * nb
        pltpu.sync_copy(idx_hbm.at[wid], idx_v)

        iota16 = lax.iota(jnp.int32, 16)
        cols_base = [iota16 + 16 * bd for bd in range(4)]

        def fire_gather(j, slot):
            pltpu.async_copy(tbl_hbm.at[idx_v.at[j]],
                             gbuf.at[slot], gsem.at[slot])

        def wait_gather(slot):
            pltpu.make_async_copy(tbl_hbm.at[idx_v.at[0]],
                                  gbuf.at[slot], gsem.at[slot]).wait()

        def fire_scatter(q, slot):
            l, c = q // cpl, q % cpl
            pltpu.async_copy(
                tbuf.at[slot],
                out_hbm.at[pl.ds(l * _D, _D), pl.ds(b_base + c * _CHUNK, _CHUNK)],
                ssem.at[slot])

        def wait_scatter(slot):
            pltpu.make_async_copy(
                tbuf.at[slot],
                out_hbm.at[pl.ds(0, _D), pl.ds(b_base, _CHUNK)],
                ssem.at[slot]).wait()

        def transpose(slot_g, slot_t):
            src = gbuf.at[slot_g]
            dst = tbuf.at[slot_t]

            def tk(kk, carry):
                rot = lax.rem(iota16 + kk, 16)
                for bb in range(8):
                    rows = rot + 16 * bb
                    for bd in range(4):
                        v = plsc.load_gather(src, [rows, cols_base[bd]])
                        plsc.store_scatter(dst, [cols_base[bd], rows], v)
                return carry

            lax.fori_loop(0, 16, tk, 0)

        def process(q, slot_g, slot_t, first_round):
            wait_gather(slot_g)
            if not first_round:
                wait_scatter(slot_t)
            transpose(slot_g, slot_t)
            fire_scatter(q, slot_t)

        for j in range(_LAG):
            fire_gather(j, j % _NGB)

        for u in range(_BLK):
            j = _LAG + u
            fire_gather(j, j % _NGB)
            process(u, u % _NGB, u % _NTB, first_round=(u < _NTB))

        def body(t, carry):
            for u in range(_BLK):
                j = _LAG + t * _BLK + u
                fire_gather(j, (_LAG + u) % _NGB)
                process(j - _LAG, u % _NGB, u % _NTB, first_round=False)
            return carry

        lax.fori_loop(1, (n_chunks - _LAG) // _BLK, body, 0)

        for q in range(n_main, n_chunks):
            j = q + _LAG
            if j < n_chunks:
                fire_gather(j, j % _NGB)
            process(q, q % _NGB, q % _NTB, first_round=False)
        for q in range(n_chunks - _NTB, n_chunks):
            wait_scatter(q % _NTB)

    return k(tbl2, idx4)


def kernel(t, table, W1, b1, W2, b2):
    tbl2 = _transform_table(table, W1, b1, W2, b2)
    B, L = t.shape
    nb = B // _NW
    cpl = nb // _CHUNK
    idx4 = (t.astype(jnp.int32).T
            .reshape(L, _NW, cpl, _CHUNK)
            .transpose(1, 0, 2, 3)
            .reshape(_NW, L * cpl, _CHUNK))
    P = _sc_gather_t(tbl2, idx4, L, B).reshape(L, _D, B)
    return lax.transpose(P, (2, 0, 1))
